# baseline (speedup 1.0000x reference)
import functools

import jax
import jax.numpy as jnp
from jax import lax
from jax.experimental import pallas as pl
from jax.experimental.pallas import tpu as pltpu
from jax.experimental.pallas import tpu_sc as plsc

N, D, K, C = 4096, 64, 512, 256
PD = 128
BLK = 4096
NB = N // BLK
CH = 32

_SC = plsc.get_sparse_core_info()
NW = _SC.num_cores * _SC.num_subcores
BPW = N // NW


def _assign_body(x_ref, p_ref, idx_ref, pp_ref, p2_ref, pn_ref):
    @pl.when(pl.program_id(0) == 0)
    def _():
        p = p_ref[...]
        p2_ref[...] = -2.0 * p
        pn_ref[...] = lax.dot_general(p * p, jnp.ones((8, D), jnp.float32),
                                      (((1,), (1,)), ((), ())),
                                      preferred_element_type=jnp.float32,
                                      precision=lax.Precision.HIGHEST)[:, :1]
        pp_ref[...] = jnp.concatenate([p, p], axis=1)

    xt = x_ref[...]
    s = lax.dot_general(p2_ref[...], xt, (((1,), (0,)), ((), ())),
                        preferred_element_type=jnp.float32,
                        precision=lax.Precision.HIGHEST)
    s = s + pn_ref[...]
    m = jnp.min(s, axis=0, keepdims=True)
    kf = lax.broadcasted_iota(jnp.int32, (K, BLK), 0).astype(jnp.float32)
    idxf = jnp.min(jnp.where(s == m, kf, float(K)), axis=0, keepdims=True)
    idx_ref[...] = idxf.astype(jnp.int32)


def _assign(x, proxies):
    return pl.pallas_call(
        _assign_body,
        grid=(NB,),
        in_specs=[pl.BlockSpec((D, BLK), lambda i: (0, i)),
                  pl.BlockSpec((K, D), lambda i: (0, 0))],
        out_specs=[pl.BlockSpec((1, BLK), lambda i: (0, i)),
                   pl.BlockSpec((K, PD), lambda i: (0, 0))],
        out_shape=[jax.ShapeDtypeStruct((1, N), jnp.int32),
                   jax.ShapeDtypeStruct((K, PD), jnp.float32)],
        scratch_shapes=[pltpu.VMEM((K, D), jnp.float32),
                        pltpu.VMEM((K, 1), jnp.float32)],
    )(x, proxies)


def _make_sc_gather():
    mesh = plsc.VectorSubcoreMesh(core_axis_name="c", subcore_axis_name="s")

    nch = BPW // CH

    @functools.partial(
        pl.kernel,
        mesh=mesh,
        out_type=[jax.ShapeDtypeStruct((N, C), jnp.float32),
                  jax.ShapeDtypeStruct((N, PD), jnp.float32)],
        scratch_types=[pltpu.VMEM((BPW,), jnp.int32),
                       pltpu.VMEM((BPW, C), jnp.float32),
                       pltpu.VMEM((BPW, PD), jnp.float32)]
                      + [pltpu.SemaphoreType.DMA] * (2 * nch + 2),
    )
    def gather_k(lab_hbm, pp_hbm, idx_hbm, outl_hbm, outp_hbm,
                 idx_v, lab_v, pp_v, *sems):
        wid = lax.axis_index("s") * _SC.num_cores + lax.axis_index("c")
        base = wid * BPW
        sem_wl, sem_wp = sems[2 * nch], sems[2 * nch + 1]
        pltpu.sync_copy(idx_hbm.at[0, pl.ds(base, BPW)], idx_v)
        gl, gp = [], []
        for c in range(nch):
            rows = pl.ds(c * CH, CH)
            ii = idx_v.at[rows]
            gl.append(pltpu.async_copy(lab_hbm.at[ii], lab_v.at[rows],
                                       sems[2 * c]))
            gp.append(pltpu.async_copy(pp_hbm.at[ii], pp_v.at[rows],
                                       sems[2 * c + 1]))
        wb = []
        for c in range(nch):
            rows = pl.ds(c * CH, CH)
            orows = pl.ds(base + c * CH, CH)
            gl[c].wait()
            wb.append(pltpu.async_copy(lab_v.at[rows], outl_hbm.at[orows],
                                       sem_wl))
            gp[c].wait()
            wb.append(pltpu.async_copy(pp_v.at[rows], outp_hbm.at[orows],
                                       sem_wp))
        for w in wb:
            w.wait()

    return gather_k


_sc_gather = _make_sc_gather()


def kernel(x, proxies, labels):
    idx, pp = _assign(jnp.transpose(x), proxies)
    lx, pxp = _sc_gather(labels, pp, idx)
    return x, pxp[:, :D], lx


# speedup vs baseline: 1.0490x; 1.0490x over previous
import functools

import jax
import jax.numpy as jnp
from jax import lax
from jax.experimental import pallas as pl
from jax.experimental.pallas import tpu as pltpu
from jax.experimental.pallas import tpu_sc as plsc

N, D, K, C = 4096, 64, 512, 256
PD = 128
BLK = 4096
NB = N // BLK
CH = 128

_SC = plsc.get_sparse_core_info()
NW = _SC.num_cores * _SC.num_subcores
BPW = N // NW


def _assign_body(x_ref, p_ref, idx_ref, pp_ref, p2_ref, pn_ref):
    @pl.when(pl.program_id(0) == 0)
    def _():
        p = p_ref[...]
        p2_ref[...] = -2.0 * p
        pn_ref[...] = lax.dot_general(p * p, jnp.ones((8, D), jnp.float32),
                                      (((1,), (1,)), ((), ())),
                                      preferred_element_type=jnp.float32,
                                      precision=lax.Precision.HIGHEST)[:, :1]
        pp_ref[...] = jnp.concatenate([p, p], axis=1)

    xt = x_ref[...]
    s = lax.dot_general(p2_ref[...], xt, (((1,), (0,)), ((), ())),
                        preferred_element_type=jnp.float32,
                        precision=lax.Precision.HIGHEST)
    s = s + pn_ref[...]
    m = jnp.min(s, axis=0, keepdims=True)
    kf = lax.broadcasted_iota(jnp.int32, (K, BLK), 0).astype(jnp.float32)
    idxf = jnp.min(jnp.where(s == m, kf, float(K)), axis=0, keepdims=True)
    idx_ref[...] = idxf.astype(jnp.int32)


def _assign(x, proxies):
    return pl.pallas_call(
        _assign_body,
        grid=(NB,),
        in_specs=[pl.BlockSpec((D, BLK), lambda i: (0, i)),
                  pl.BlockSpec((K, D), lambda i: (0, 0))],
        out_specs=[pl.BlockSpec((1, BLK), lambda i: (0, i)),
                   pl.BlockSpec((K, PD), lambda i: (0, 0))],
        out_shape=[jax.ShapeDtypeStruct((1, N), jnp.int32),
                   jax.ShapeDtypeStruct((K, PD), jnp.float32)],
        scratch_shapes=[pltpu.VMEM((K, D), jnp.float32),
                        pltpu.VMEM((K, 1), jnp.float32)],
    )(x, proxies)


def _make_sc_gather():
    mesh = plsc.VectorSubcoreMesh(core_axis_name="c", subcore_axis_name="s")

    nch = BPW // CH

    @functools.partial(
        pl.kernel,
        mesh=mesh,
        out_type=[jax.ShapeDtypeStruct((N, C), jnp.float32),
                  jax.ShapeDtypeStruct((N, PD), jnp.float32)],
        scratch_types=[pltpu.VMEM((BPW,), jnp.int32),
                       pltpu.VMEM((BPW, C), jnp.float32),
                       pltpu.VMEM((BPW, PD), jnp.float32)]
                      + [pltpu.SemaphoreType.DMA] * (2 * nch + 2),
    )
    def gather_k(lab_hbm, pp_hbm, idx_hbm, outl_hbm, outp_hbm,
                 idx_v, lab_v, pp_v, *sems):
        wid = lax.axis_index("s") * _SC.num_cores + lax.axis_index("c")
        base = wid * BPW
        sem_wl, sem_wp = sems[2 * nch], sems[2 * nch + 1]
        pltpu.sync_copy(idx_hbm.at[0, pl.ds(base, BPW)], idx_v)
        gl, gp = [], []
        for c in range(nch):
            rows = pl.ds(c * CH, CH)
            ii = idx_v.at[rows]
            gl.append(pltpu.async_copy(lab_hbm.at[ii], lab_v.at[rows],
                                       sems[2 * c]))
            gp.append(pltpu.async_copy(pp_hbm.at[ii], pp_v.at[rows],
                                       sems[2 * c + 1]))
        wb = []
        for c in range(nch):
            rows = pl.ds(c * CH, CH)
            orows = pl.ds(base + c * CH, CH)
            gl[c].wait()
            wb.append(pltpu.async_copy(lab_v.at[rows], outl_hbm.at[orows],
                                       sem_wl))
            gp[c].wait()
            wb.append(pltpu.async_copy(pp_v.at[rows], outp_hbm.at[orows],
                                       sem_wp))
        for w in wb:
            w.wait()

    return gather_k


_sc_gather = _make_sc_gather()


def kernel(x, proxies, labels):
    idx, pp = _assign(jnp.transpose(x), proxies)
    lx, pxp = _sc_gather(labels, pp, idx)
    return x, pxp[:, :D], lx
